# Initial kernel scaffold; baseline (speedup 1.0000x reference)
import jax
import jax.numpy as jnp
from jax.experimental import pallas as pl


def kernel(world_coords, grid0, grid1):
    raise NotImplementedError("write your pallas kernel here")


# speedup vs baseline: 1.5555x; 1.5555x over previous
import functools

import jax
import jax.numpy as jnp
from jax import lax
from jax.experimental import pallas as pl
from jax.experimental.pallas import tpu as pltpu
from jax.experimental.pallas import tpu_sc as plsc

_N = 131072
_C = 32
_RES0 = 64
_RES1 = 128
_SCENE_SIZE = 508.0
_NC = 2
_NS = 16
_NW = _NC * _NS
_P = _N // _NW
_CB = 16
_CHUNKS = _P // _CB


def _corner_data(xw, yw, zw, res):
    rm1 = float(res - 1)

    def axis(w):
        gc = ((w - 0.0) / _SCENE_SIZE) * rm1
        v = ((gc + 1.0) * 0.5) * rm1
        v = jnp.minimum(jnp.maximum(v, 0.0), rm1)
        i0 = v.astype(jnp.int32)
        frac = v - i0.astype(jnp.float32)
        i1 = jnp.minimum(i0 + 1, res - 1)
        return i0, i1, frac

    x0, x1, fx = axis(xw)
    y0, y1, fy = axis(yw)
    z0, z1, fz = axis(zw)

    gx0 = 1.0 - fx
    gy0 = 1.0 - fy
    gz0 = 1.0 - fz
    wzy = [gz0 * gy0, gz0 * fy, fz * gy0, fz * fy]
    wts = []
    for zy in range(4):
        wts.append(wzy[zy] * gx0)
        wts.append(wzy[zy] * fx)

    zb0 = z0 * (res * res)
    zb1 = z1 * (res * res)
    yo0 = y0 * res
    yo1 = y1 * res
    zy_base = [zb0 + yo0, zb0 + yo1, zb1 + yo0, zb1 + yo1]
    idx = []
    for zy in range(4):
        idx.append(zy_base[zy] + x0)
        idx.append(zy_base[zy] + x1)
    return wts, idx


_GATHER_DN = lax.GatherDimensionNumbers(
    offset_dims=(), collapsed_slice_dims=(0,), start_index_map=(0,))


def _splat(vec, pidx):
    return lax.gather(vec, pidx[:, None], _GATHER_DN, (1,),
                      mode=lax.GatherScatterMode.PROMISE_IN_BOUNDS)


def _blend(rows_ref, out_ref, wts, col_off):
    for p in range(_CB):
        pidx = jnp.full((16,), p, jnp.int32)
        ws = [_splat(w, pidx) for w in wts]
        for h in range(2):
            sl = pl.ds(h * 16, 16)
            acc = ws[0] * rows_ref[p, sl]
            for c in range(1, 8):
                acc = acc + ws[c] * rows_ref[c * _CB + p, sl]
            out_ref[p, pl.ds(col_off + h * 16, 16)] = acc


def _make_sc_kernel():
    mesh = plsc.VectorSubcoreMesh(core_axis_name="c", subcore_axis_name="s")

    @functools.partial(
        pl.kernel,
        mesh=mesh,
        out_type=jax.ShapeDtypeStruct((_N, 2 * _C), jnp.float32),
        compiler_params=pltpu.CompilerParams(use_tc_tiling_on_sc=False),
        scratch_types=[
            pltpu.VMEM((_P,), jnp.float32),
            pltpu.VMEM((_P,), jnp.float32),
            pltpu.VMEM((_P,), jnp.float32),
            pltpu.VMEM((8 * _CB,), jnp.int32),
            pltpu.VMEM((8 * _CB,), jnp.int32),
            pltpu.VMEM((8 * _CB, _C), jnp.float32),
            pltpu.VMEM((8 * _CB, _C), jnp.float32),
            pltpu.VMEM((_CB, 2 * _C), jnp.float32),
            pltpu.SemaphoreType.DMA,
            pltpu.SemaphoreType.DMA,
        ],
    )
    def sc_kernel(xs_hbm, ys_hbm, zs_hbm, t0_hbm, t1_hbm, out_hbm,
                  xv, yv, zv, idx0, idx1, rows0, rows1, outv, sem0, sem1):
        wid = lax.axis_index("s") * _NC + lax.axis_index("c")
        base = wid * _P
        pltpu.sync_copy(xs_hbm.at[pl.ds(base, _P)], xv)
        pltpu.sync_copy(ys_hbm.at[pl.ds(base, _P)], yv)
        pltpu.sync_copy(zs_hbm.at[pl.ds(base, _P)], zv)

        def chunk(u, carry):
            off = u * _CB
            xw = xv[pl.ds(off, _CB)]
            yw = yv[pl.ds(off, _CB)]
            zw = zv[pl.ds(off, _CB)]
            wts0, ind0 = _corner_data(xw, yw, zw, _RES0)
            wts1, ind1 = _corner_data(xw, yw, zw, _RES1)
            for c in range(8):
                idx0[pl.ds(c * 16, 16)] = ind0[c]
                idx1[pl.ds(c * 16, 16)] = ind1[c]
            cp0 = pltpu.async_copy(t0_hbm.at[idx0], rows0, sem0)
            cp1 = pltpu.async_copy(t1_hbm.at[idx1], rows1, sem1)
            cp0.wait()
            _blend(rows0, outv, wts0, 0)
            cp1.wait()
            _blend(rows1, outv, wts1, _C)
            pltpu.sync_copy(outv, out_hbm.at[pl.ds(base + off, _CB)])
            return carry

        lax.fori_loop(0, _CHUNKS, chunk, 0)

    return sc_kernel


_SC_KERNEL = _make_sc_kernel()


def kernel(world_coords, grid0, grid1):
    ct = world_coords.T
    t0 = grid0.reshape(-1, _C)
    t1 = grid1.reshape(-1, _C)
    return _SC_KERNEL(ct[0], ct[1], ct[2], t0, t1)


# speedup vs baseline: 1.6550x; 1.0640x over previous
import functools

import jax
import jax.numpy as jnp
from jax import lax
from jax.experimental import pallas as pl
from jax.experimental.pallas import tpu as pltpu
from jax.experimental.pallas import tpu_sc as plsc

_N = 131072
_C = 32
_RES0 = 64
_RES1 = 128
_SCENE_SIZE = 508.0
_NC = 2
_NS = 16
_NW = _NC * _NS
_P = _N // _NW
_CB = 16
_CHUNKS = _P // _CB
_NBUF = 2
_GROUPS = _CHUNKS // _NBUF
_R = 8 * _CB


def _axis_coord(w, res):
    rm1 = float(res - 1)
    gc = ((w - 0.0) / _SCENE_SIZE) * rm1
    v = ((gc + 1.0) * 0.5) * rm1
    return jnp.minimum(jnp.maximum(v, 0.0), rm1)


_GATHER_DN = lax.GatherDimensionNumbers(
    offset_dims=(), collapsed_slice_dims=(0,), start_index_map=(0,))


def _splat(vec, pidx):
    return lax.gather(vec, pidx[:, None], _GATHER_DN, (1,),
                      mode=lax.GatherScatterMode.PROMISE_IN_BOUNDS)


def _blend(rows_ref, out_ref, row_off, fx, fy, fz, col_off):
    for p in range(_CB):
        pidx = jnp.full((16,), p, jnp.int32)
        wx = _splat(fx, pidx)
        wy = _splat(fy, pidx)
        wz = _splat(fz, pidx)
        for h in range(2):
            sl = pl.ds(h * 16, 16)
            c = [rows_ref[k * _CB + p, sl] for k in range(8)]
            t00 = c[0] + wx * (c[1] - c[0])
            t01 = c[2] + wx * (c[3] - c[2])
            t10 = c[4] + wx * (c[5] - c[4])
            t11 = c[6] + wx * (c[7] - c[6])
            top = t00 + wy * (t01 - t00)
            bot = t10 + wy * (t11 - t10)
            res = top + wz * (bot - top)
            out_ref[row_off + p, pl.ds(col_off + h * 16, 16)] = res


def _make_sc_kernel():
    mesh = plsc.VectorSubcoreMesh(core_axis_name="c", subcore_axis_name="s")

    @functools.partial(
        pl.kernel,
        mesh=mesh,
        out_type=jax.ShapeDtypeStruct((_N, 2 * _C), jnp.float32),
        compiler_params=pltpu.CompilerParams(use_tc_tiling_on_sc=False),
        scratch_types=[
            pltpu.VMEM((_P,), jnp.float32),
            pltpu.VMEM((_P,), jnp.float32),
            pltpu.VMEM((_P,), jnp.float32),
            [pltpu.VMEM((_P,), jnp.float32) for _ in range(3)],
            [pltpu.VMEM((_P,), jnp.float32) for _ in range(3)],
            [pltpu.VMEM((_R,), jnp.int32) for _ in range(_NBUF)],
            [pltpu.VMEM((_R,), jnp.int32) for _ in range(_NBUF)],
            [pltpu.VMEM((_R, _C), jnp.float32) for _ in range(_NBUF)],
            [pltpu.VMEM((_R, _C), jnp.float32) for _ in range(_NBUF)],
            pltpu.VMEM((_NBUF * _CB, 2 * _C), jnp.float32),
            [pltpu.SemaphoreType.DMA for _ in range(_NBUF)],
            [pltpu.SemaphoreType.DMA for _ in range(_NBUF)],
        ],
    )
    def sc_kernel(xs_hbm, ys_hbm, zs_hbm, t0_hbm, t1_hbm, out_hbm,
                  xv, yv, zv, v0, v1, idx0, idx1, rows0, rows1, outstg,
                  sem0, sem1):
        wid = lax.axis_index("s") * _NC + lax.axis_index("c")
        base = wid * _P
        pltpu.sync_copy(xs_hbm.at[pl.ds(base, _P)], xv)
        pltpu.sync_copy(ys_hbm.at[pl.ds(base, _P)], yv)
        pltpu.sync_copy(zs_hbm.at[pl.ds(base, _P)], zv)

        def coord_body(u, carry):
            off = u * _CB
            sl = pl.ds(off, _CB)
            w3 = (xv[sl], yv[sl], zv[sl])
            for a in range(3):
                v0[a][sl] = _axis_coord(w3[a], _RES0)
                v1[a][sl] = _axis_coord(w3[a], _RES1)
            return carry

        lax.fori_loop(0, _CHUNKS, coord_body, 0)

        def fire(c, b):
            off = c * _CB
            sl = pl.ds(off, _CB)
            for lvl, res, ibuf, tbl, rbuf, sem in (
                    (v0, _RES0, idx0[b], t0_hbm, rows0[b], sem0[b]),
                    (v1, _RES1, idx1[b], t1_hbm, rows1[b], sem1[b])):
                def bounds(v):
                    i0 = v.astype(jnp.int32)
                    return i0, jnp.minimum(i0 + 1, res - 1)
                x0, x1 = bounds(lvl[0][sl])
                y0, y1 = bounds(lvl[1][sl])
                z0, z1 = bounds(lvl[2][sl])
                zb0 = z0 * (res * res)
                zb1 = z1 * (res * res)
                yo0 = y0 * res
                yo1 = y1 * res
                zy = (zb0 + yo0, zb0 + yo1, zb1 + yo0, zb1 + yo1)
                k = 0
                for zyb in zy:
                    ibuf[pl.ds(k * _CB, _CB)] = zyb + x0
                    ibuf[pl.ds((k + 1) * _CB, _CB)] = zyb + x1
                    k += 2
                pltpu.async_copy(tbl.at[ibuf], rbuf, sem)

        def wait(b):
            pltpu.make_async_copy(t0_hbm.at[idx0[b]], rows0[b], sem0[b]).wait()
            pltpu.make_async_copy(t1_hbm.at[idx1[b]], rows1[b], sem1[b]).wait()

        def fracs(c, lvl):
            sl = pl.ds(c * _CB, _CB)
            out = []
            for a in range(3):
                v = lvl[a][sl]
                out.append(v - v.astype(jnp.int32).astype(jnp.float32))
            return out

        fire(0, 0)

        def group(g, carry):
            for b in range(_NBUF):
                c = g * _NBUF + b
                wait(b)
                cn = jnp.minimum(c + 1, _CHUNKS - 1)
                fire(cn, 1 - b)
                fx0, fy0, fz0 = fracs(c, v0)
                _blend(rows0[b], outstg, b * _CB, fx0, fy0, fz0, 0)
                fx1, fy1, fz1 = fracs(c, v1)
                _blend(rows1[b], outstg, b * _CB, fx1, fy1, fz1, _C)
            pltpu.sync_copy(outstg, out_hbm.at[pl.ds(base + g * (_NBUF * _CB), _NBUF * _CB)])
            return carry

        lax.fori_loop(0, _GROUPS, group, 0)

        wait(0)

    return sc_kernel


_SC_KERNEL = _make_sc_kernel()


def kernel(world_coords, grid0, grid1):
    ct = world_coords.T
    t0 = grid0.reshape(-1, _C)
    t1 = grid1.reshape(-1, _C)
    return _SC_KERNEL(ct[0], ct[1], ct[2], t0, t1)
